# baseline (speedup 1.0000x reference)
import functools

import jax
import jax.numpy as jnp
from jax import lax
from jax.experimental import pallas as pl
from jax.experimental.pallas import tpu as pltpu
from jax.experimental.pallas import tpu_sc as plsc

N = 10000
E = 320000
D = 128
H = 128
L = 64
B = 16

NP = 10240
NTILES = 32
EPW = E // NTILES
CH = 80
NCH = EPW // CH
ECH = 80
ENCH = EPW // ECH
RPT = NP // 16

_MESH = plsc.VectorSubcoreMesh(core_axis_name="c", subcore_axis_name="s")


def _sc_degree(dst3, ones_c, zeros_r):
    @functools.partial(
        pl.kernel,
        mesh=_MESH,
        out_type=jax.ShapeDtypeStruct((2, NP), jnp.float32),
        scratch_types=[
            pltpu.VMEM((NCH, CH), jnp.int32),
            pltpu.VMEM((CH,), jnp.float32),
            pltpu.VMEM_SHARED((NP,), jnp.float32),
            pltpu.SemaphoreType.DMA,
            pltpu.SemaphoreType.DMA,
        ],
    )
    def k(dst_hbm, ones_hbm, zeros_hbm, out_hbm, idx_v, ones_v, acc_sh,
          sem0, sem1):
        c = lax.axis_index("c")
        s = lax.axis_index("s")
        w = c * 16 + s
        pltpu.sync_copy(ones_hbm, ones_v)
        pltpu.sync_copy(dst_hbm.at[w], idx_v)
        pltpu.sync_copy(zeros_hbm, acc_sh.at[pl.ds(s * RPT, RPT)])
        plsc.subcore_barrier()

        def scat(i, sem):
            pltpu.async_copy(ones_v, acc_sh.at[idx_v.at[i]], sem, add=True)

        def wait(i, sem):
            pltpu.make_async_copy(ones_v, acc_sh.at[idx_v.at[i]], sem).wait()

        scat(0, sem0)
        scat(1, sem1)

        def body(g, carry):
            i = 2 * g
            wait(i, sem0)
            scat(i + 2, sem0)
            wait(i + 1, sem1)
            scat(i + 3, sem1)
            return carry

        lax.fori_loop(0, (NCH - 3) // 2, body, 0)
        wait(NCH - 3, sem0)
        scat(NCH - 1, sem0)
        wait(NCH - 2, sem1)
        wait(NCH - 1, sem0)
        plsc.subcore_barrier()
        pltpu.sync_copy(acc_sh.at[pl.ds(s * RPT, RPT)],
                        out_hbm.at[c, pl.ds(s * RPT, RPT)])

    return k(dst3, ones_c, zeros_r)


def _sc_edge_accum(eidx4, y, zeros_rows):
    NB = 4
    K = NB // 2
    NI = NB + K
    GS = (ENCH - K) // NB

    @functools.partial(
        pl.kernel,
        mesh=_MESH,
        out_type=jax.ShapeDtypeStruct((2, NP, H), jnp.float32),
        scratch_types=(
            [pltpu.VMEM((2, ECH), jnp.int32) for _ in range(NI)]
            + [pltpu.VMEM((ECH, H), jnp.float32) for _ in range(NB)]
            + [pltpu.VMEM_SHARED((NP, H), jnp.float32)]
            + [pltpu.SemaphoreType.DMA] * (2 * NB + NI)
        ),
    )
    def k(eidx_hbm, y_hbm, zeros_hbm, out_hbm, *refs):
        idx = refs[0:NI]
        rows = refs[NI:NI + NB]
        acc_sh = refs[NI + NB]
        gsem = refs[NI + NB + 1:NI + 2 * NB + 1]
        scsem = refs[NI + 2 * NB + 1:NI + 3 * NB + 1]
        isem = refs[NI + 3 * NB + 1:2 * NI + 3 * NB + 1]
        c = lax.axis_index("c")
        s = lax.axis_index("s")
        w = c * 16 + s

        def load_idx(cd, m):
            pltpu.async_copy(eidx_hbm.at[w, cd], idx[m], isem[m])

        def wait_idx(cd, m):
            pltpu.make_async_copy(eidx_hbm.at[w, cd], idx[m], isem[m]).wait()

        def gather(j, m):
            pltpu.async_copy(y_hbm.at[idx[m].at[0]], rows[j], gsem[j])

        def wait_gather(j, m):
            pltpu.make_async_copy(
                y_hbm.at[idx[m].at[0]], rows[j], gsem[j]).wait()

        def scatter(j, m):
            pltpu.async_copy(rows[j], acc_sh.at[idx[m].at[1]],
                             scsem[j], add=True)

        def wait_scatter(j, m):
            pltpu.make_async_copy(
                rows[j], acc_sh.at[idx[m].at[1]], scsem[j]).wait()

        def step(cd, rr, do_scwait, do_load=True, do_gather=True):
            j = rr % NB
            wait_gather(j, rr % NI)
            scatter(j, rr % NI)
            jj = (j + K) % NB
            if do_scwait:
                wait_scatter(jj, (rr - K) % NI)
            if do_load:
                load_idx(cd + NB, (rr + NB) % NI)
            if do_gather:
                wait_idx(cd + K, (rr + K) % NI)
                gather(jj, (rr + K) % NI)

        for cc in range(NB):
            load_idx(cc, cc % NI)
        for j in range(K):
            wait_idx(j, j % NI)
            gather(j, j % NI)
        pltpu.sync_copy(zeros_hbm, acc_sh.at[pl.ds(s * RPT, RPT)])
        plsc.subcore_barrier()
        for cc in range(NB):
            step(cc, cc, do_scwait=cc >= K)

        U = 12
        T = (ENCH - K - NB) // U

        def body(h, carry):
            base = U * h + NB
            for r_off in range(U):
                step(base + r_off, NB + r_off, do_scwait=True)
            return carry

        lax.fori_loop(0, T, body, 0)

        for cc in range(NB + U * T, ENCH):
            if cc + K < ENCH:
                step(cc, cc, do_scwait=True, do_load=cc + NB < ENCH)
            else:
                wait_gather(cc % NB, cc % NI)
                scatter(cc % NB, cc % NI)
        for t in range(NB):
            cc = ENCH - NB + t
            wait_scatter(cc % NB, cc % NI)

        plsc.subcore_barrier()
        pltpu.sync_copy(acc_sh.at[pl.ds(s * RPT, RPT)],
                        out_hbm.at[c, pl.ds(s * RPT, RPT)])

    return k(eidx4, y, zeros_rows)


_RB = 1000
_NG = N // _RB


def _dinv_block(degp_b):
    return lax.rsqrt(degp_b[0] + degp_b[1] + 1.0)


def _tc_scale_matmul(x, W, degp3):
    def body(x_b, w_b, degp_b, y_b):
        dinv = _dinv_block(degp_b)
        y_b[...] = jnp.dot(x_b[...], w_b[...],
                           preferred_element_type=jnp.float32) * dinv

    return pl.pallas_call(
        body,
        grid=(_NG,),
        in_specs=[
            pl.BlockSpec((_RB, D), lambda i: (i, 0)),
            pl.BlockSpec((D, H), lambda i: (0, 0)),
            pl.BlockSpec((2, _RB, 1), lambda i: (0, i, 0)),
        ],
        out_specs=pl.BlockSpec((_RB, H), lambda i: (i, 0)),
        out_shape=jax.ShapeDtypeStruct((N, H), jnp.float32),
    )(x, W, degp3)


def _tc_conv1_out(acc, y1, degp3, b1, W2):
    def body(acc_b, y1_b, degp_b, b1_b, w2_b, y2_b):
        dinv = _dinv_block(degp_b)
        h = jnp.maximum(dinv * (acc_b[0] + acc_b[1] + y1_b[...]) + b1_b[...],
                        0.0)
        y2_b[...] = jnp.dot(h, w2_b[...],
                            preferred_element_type=jnp.float32) * dinv

    return pl.pallas_call(
        body,
        grid=(_NG,),
        in_specs=[
            pl.BlockSpec((2, _RB, H), lambda i: (0, i, 0)),
            pl.BlockSpec((_RB, H), lambda i: (i, 0)),
            pl.BlockSpec((2, _RB, 1), lambda i: (0, i, 0)),
            pl.BlockSpec((1, H), lambda i: (0, 0)),
            pl.BlockSpec((H, H), lambda i: (0, 0)),
        ],
        out_specs=pl.BlockSpec((_RB, H), lambda i: (i, 0)),
        out_shape=jax.ShapeDtypeStruct((N, H), jnp.float32),
    )(acc, y1, degp3, b1, W2)


def _tc_conv2_pool(acc, y2, degp3, b2, Wg, bg, batch2d, Wm, bm):
    def body(acc_r, y2_r, degp_r, b2_r, wg_r, bg_r, batch_r, wm_r, bm_r,
             mu_r):
        dinv = lax.rsqrt(degp_r[0, 0:N, :] + degp_r[1, 0:N, :] + 1.0)
        h2 = (dinv * (acc_r[0, 0:N, :] + acc_r[1, 0:N, :] + y2_r[...])
              + b2_r[...])
        g = jnp.dot(h2, wg_r[...],
                    preferred_element_type=jnp.float32) + bg_r[...]
        mask = (batch_r[...] ==
                lax.broadcasted_iota(jnp.int32, (N, B), 1))
        maskf = mask.astype(jnp.float32)
        gm = jnp.max(jnp.where(mask, g, -1e30), axis=0, keepdims=True)
        gmax_node = jnp.sum(maskf * gm, axis=1, keepdims=True)
        ge = jnp.exp(g - gmax_node)
        gs = jnp.sum(maskf * ge, axis=0, keepdims=True)
        gs_node = jnp.sum(maskf * gs, axis=1, keepdims=True)
        gate = ge / gs_node
        wgt = maskf * gate
        pooled = lax.dot_general(wgt, h2,
                                 (((0,), (0,)), ((), ())),
                                 preferred_element_type=jnp.float32)
        mu_r[...] = jnp.dot(pooled, wm_r[...],
                            preferred_element_type=jnp.float32) + bm_r[...]

    return pl.pallas_call(
        body,
        in_specs=[
            pl.BlockSpec((2, NP, H), lambda: (0, 0, 0)),
            pl.BlockSpec((N, H), lambda: (0, 0)),
            pl.BlockSpec((2, NP, 1), lambda: (0, 0, 0)),
            pl.BlockSpec((1, H), lambda: (0, 0)),
            pl.BlockSpec((H, 1), lambda: (0, 0)),
            pl.BlockSpec((1, 1), lambda: (0, 0)),
            pl.BlockSpec((N, 1), lambda: (0, 0)),
            pl.BlockSpec((H, L), lambda: (0, 0)),
            pl.BlockSpec((1, L), lambda: (0, 0)),
        ],
        out_specs=pl.BlockSpec((B, L), lambda: (0, 0)),
        out_shape=jax.ShapeDtypeStruct((B, L), jnp.float32),
    )(acc, y2, degp3, b2, Wg, bg, batch2d, Wm, bm)


def kernel(x, edge_index, batch, W1, b1, W2, b2, Wg, bg, Wm, bm, Wv, bv):
    del Wv, bv
    dst3 = edge_index[1].reshape(NTILES, NCH, CH)
    eidx4 = jnp.stack([edge_index[0].reshape(NTILES, ENCH, ECH),
                       edge_index[1].reshape(NTILES, ENCH, ECH)],
                      axis=2)

    ones_c = jnp.ones((CH,), jnp.float32)
    zeros_r = jnp.zeros((RPT,), jnp.float32)
    zeros_rows = jnp.zeros((RPT, H), jnp.float32)

    degp = _sc_degree(dst3, ones_c, zeros_r)
    degp3 = degp.reshape(2, NP, 1)

    y1 = _tc_scale_matmul(x, W1, degp3)
    acc1 = _sc_edge_accum(eidx4, y1, zeros_rows)
    y2 = _tc_conv1_out(acc1, y1, degp3,
                       b1.reshape(1, H), W2)
    acc2 = _sc_edge_accum(eidx4, y2, zeros_rows)
    mu = _tc_conv2_pool(acc2, y2, degp3, b2.reshape(1, H),
                        Wg, bg.reshape(1, 1),
                        batch.reshape(N, 1), Wm, bm.reshape(1, L))
    return mu


# speedup vs baseline: 1.0008x; 1.0008x over previous
import functools

import jax
import jax.numpy as jnp
from jax import lax
from jax.experimental import pallas as pl
from jax.experimental.pallas import tpu as pltpu
from jax.experimental.pallas import tpu_sc as plsc

N = 10000
E = 320000
D = 128
H = 128
L = 64
B = 16

NP = 10240
NTILES = 32
EPW = E // NTILES
CH = 80
NCH = EPW // CH
ECH = 80
ENCH = EPW // ECH
RPT = NP // 16

_MESH = plsc.VectorSubcoreMesh(core_axis_name="c", subcore_axis_name="s")


def _sc_degree(dst3, ones_c, zeros_r):
    @functools.partial(
        pl.kernel,
        mesh=_MESH,
        out_type=jax.ShapeDtypeStruct((2, NP), jnp.float32),
        scratch_types=[
            pltpu.VMEM((NCH, CH), jnp.int32),
            pltpu.VMEM((CH,), jnp.float32),
            pltpu.VMEM_SHARED((NP,), jnp.float32),
            pltpu.SemaphoreType.DMA,
            pltpu.SemaphoreType.DMA,
            pltpu.SemaphoreType.DMA,
            pltpu.SemaphoreType.DMA,
        ],
    )
    def k(dst_hbm, ones_hbm, zeros_hbm, out_hbm, idx_v, ones_v, acc_sh,
          sem0, sem1, sem2, sem3):
        c = lax.axis_index("c")
        s = lax.axis_index("s")
        w = c * 16 + s
        pltpu.sync_copy(ones_hbm, ones_v)
        pltpu.sync_copy(dst_hbm.at[w], idx_v)
        pltpu.sync_copy(zeros_hbm, acc_sh.at[pl.ds(s * RPT, RPT)])
        plsc.subcore_barrier()

        def scat(i, sem):
            pltpu.async_copy(ones_v, acc_sh.at[idx_v.at[i]], sem, add=True)

        def wait(i, sem):
            pltpu.make_async_copy(ones_v, acc_sh.at[idx_v.at[i]], sem).wait()

        sems = (sem0, sem1, sem2, sem3)
        for j in range(4):
            scat(j, sems[j])

        def body(g, carry):
            i = 4 * g
            for j in range(4):
                wait(i + j - 4, sems[j])
                scat(i + j, sems[j])
            return carry

        lax.fori_loop(1, (NCH - 1) // 4, body, 0)
        wait(NCH - 5, sem0)
        scat(NCH - 1, sem0)
        for j, i in ((1, NCH - 4), (2, NCH - 3), (3, NCH - 2), (0, NCH - 1)):
            wait(i, sems[j])
        plsc.subcore_barrier()
        pltpu.sync_copy(acc_sh.at[pl.ds(s * RPT, RPT)],
                        out_hbm.at[c, pl.ds(s * RPT, RPT)])

    return k(dst3, ones_c, zeros_r)


def _sc_edge_accum(eidx4, y, zeros_rows):
    NB = 4
    K = NB // 2
    NI = NB + K
    GS = (ENCH - K) // NB

    @functools.partial(
        pl.kernel,
        mesh=_MESH,
        out_type=jax.ShapeDtypeStruct((2, NP, H), jnp.float32),
        scratch_types=(
            [pltpu.VMEM((2, ECH), jnp.int32) for _ in range(NI)]
            + [pltpu.VMEM((ECH, H), jnp.float32) for _ in range(NB)]
            + [pltpu.VMEM_SHARED((NP, H), jnp.float32)]
            + [pltpu.SemaphoreType.DMA] * (2 * NB + NI)
        ),
    )
    def k(eidx_hbm, y_hbm, zeros_hbm, out_hbm, *refs):
        idx = refs[0:NI]
        rows = refs[NI:NI + NB]
        acc_sh = refs[NI + NB]
        gsem = refs[NI + NB + 1:NI + 2 * NB + 1]
        scsem = refs[NI + 2 * NB + 1:NI + 3 * NB + 1]
        isem = refs[NI + 3 * NB + 1:2 * NI + 3 * NB + 1]
        c = lax.axis_index("c")
        s = lax.axis_index("s")
        w = c * 16 + s

        def load_idx(cd, m):
            pltpu.async_copy(eidx_hbm.at[w, cd], idx[m], isem[m])

        def wait_idx(cd, m):
            pltpu.make_async_copy(eidx_hbm.at[w, cd], idx[m], isem[m]).wait()

        def gather(j, m):
            pltpu.async_copy(y_hbm.at[idx[m].at[0]], rows[j], gsem[j])

        def wait_gather(j, m):
            pltpu.make_async_copy(
                y_hbm.at[idx[m].at[0]], rows[j], gsem[j]).wait()

        def scatter(j, m):
            pltpu.async_copy(rows[j], acc_sh.at[idx[m].at[1]],
                             scsem[j], add=True)

        def wait_scatter(j, m):
            pltpu.make_async_copy(
                rows[j], acc_sh.at[idx[m].at[1]], scsem[j]).wait()

        def step(cd, rr, do_scwait, do_load=True, do_gather=True):
            j = rr % NB
            wait_gather(j, rr % NI)
            scatter(j, rr % NI)
            jj = (j + K) % NB
            if do_scwait:
                wait_scatter(jj, (rr - K) % NI)
            if do_load:
                load_idx(cd + NB, (rr + NB) % NI)
            if do_gather:
                wait_idx(cd + K, (rr + K) % NI)
                gather(jj, (rr + K) % NI)

        for cc in range(NB):
            load_idx(cc, cc % NI)
        for j in range(K):
            wait_idx(j, j % NI)
            gather(j, j % NI)
        pltpu.sync_copy(zeros_hbm, acc_sh.at[pl.ds(s * RPT, RPT)])
        plsc.subcore_barrier()
        for cc in range(NB):
            step(cc, cc, do_scwait=cc >= K)

        U = 12
        T = (ENCH - K - NB) // U

        def body(h, carry):
            base = U * h + NB
            for r_off in range(U):
                step(base + r_off, NB + r_off, do_scwait=True)
            return carry

        lax.fori_loop(0, T, body, 0)

        for cc in range(NB + U * T, ENCH):
            if cc + K < ENCH:
                step(cc, cc, do_scwait=True, do_load=cc + NB < ENCH)
            else:
                wait_gather(cc % NB, cc % NI)
                scatter(cc % NB, cc % NI)
        for t in range(NB):
            cc = ENCH - NB + t
            wait_scatter(cc % NB, cc % NI)

        plsc.subcore_barrier()
        pltpu.sync_copy(acc_sh.at[pl.ds(s * RPT, RPT)],
                        out_hbm.at[c, pl.ds(s * RPT, RPT)])

    return k(eidx4, y, zeros_rows)


_RB = 1000
_NG = N // _RB


def _dinv_block(degp_b):
    return lax.rsqrt(degp_b[0] + degp_b[1] + 1.0)


def _tc_scale_matmul(x, W, degp3):
    def body(x_b, w_b, degp_b, y_b):
        dinv = _dinv_block(degp_b)
        y_b[...] = jnp.dot(x_b[...], w_b[...],
                           preferred_element_type=jnp.float32) * dinv

    return pl.pallas_call(
        body,
        grid=(_NG,),
        in_specs=[
            pl.BlockSpec((_RB, D), lambda i: (i, 0)),
            pl.BlockSpec((D, H), lambda i: (0, 0)),
            pl.BlockSpec((2, _RB, 1), lambda i: (0, i, 0)),
        ],
        out_specs=pl.BlockSpec((_RB, H), lambda i: (i, 0)),
        out_shape=jax.ShapeDtypeStruct((N, H), jnp.float32),
    )(x, W, degp3)


def _tc_conv1_out(acc, y1, degp3, b1, W2):
    def body(acc_b, y1_b, degp_b, b1_b, w2_b, y2_b):
        dinv = _dinv_block(degp_b)
        h = jnp.maximum(dinv * (acc_b[0] + acc_b[1] + y1_b[...]) + b1_b[...],
                        0.0)
        y2_b[...] = jnp.dot(h, w2_b[...],
                            preferred_element_type=jnp.float32) * dinv

    return pl.pallas_call(
        body,
        grid=(_NG,),
        in_specs=[
            pl.BlockSpec((2, _RB, H), lambda i: (0, i, 0)),
            pl.BlockSpec((_RB, H), lambda i: (i, 0)),
            pl.BlockSpec((2, _RB, 1), lambda i: (0, i, 0)),
            pl.BlockSpec((1, H), lambda i: (0, 0)),
            pl.BlockSpec((H, H), lambda i: (0, 0)),
        ],
        out_specs=pl.BlockSpec((_RB, H), lambda i: (i, 0)),
        out_shape=jax.ShapeDtypeStruct((N, H), jnp.float32),
    )(acc, y1, degp3, b1, W2)


def _tc_conv2_pool(acc, y2, degp3, b2, Wg, bg, batch2d, Wm, bm):
    def body(acc_r, y2_r, degp_r, b2_r, wg_r, bg_r, batch_r, wm_r, bm_r,
             mu_r):
        dinv = lax.rsqrt(degp_r[0, 0:N, :] + degp_r[1, 0:N, :] + 1.0)
        h2 = (dinv * (acc_r[0, 0:N, :] + acc_r[1, 0:N, :] + y2_r[...])
              + b2_r[...])
        g = jnp.dot(h2, wg_r[...],
                    preferred_element_type=jnp.float32) + bg_r[...]
        mask = (batch_r[...] ==
                lax.broadcasted_iota(jnp.int32, (N, B), 1))
        maskf = mask.astype(jnp.float32)
        gm = jnp.max(jnp.where(mask, g, -1e30), axis=0, keepdims=True)
        gmax_node = jnp.sum(maskf * gm, axis=1, keepdims=True)
        ge = jnp.exp(g - gmax_node)
        gs = jnp.sum(maskf * ge, axis=0, keepdims=True)
        gs_node = jnp.sum(maskf * gs, axis=1, keepdims=True)
        gate = ge / gs_node
        wgt = maskf * gate
        pooled = lax.dot_general(wgt, h2,
                                 (((0,), (0,)), ((), ())),
                                 preferred_element_type=jnp.float32)
        mu_r[...] = jnp.dot(pooled, wm_r[...],
                            preferred_element_type=jnp.float32) + bm_r[...]

    return pl.pallas_call(
        body,
        in_specs=[
            pl.BlockSpec((2, NP, H), lambda: (0, 0, 0)),
            pl.BlockSpec((N, H), lambda: (0, 0)),
            pl.BlockSpec((2, NP, 1), lambda: (0, 0, 0)),
            pl.BlockSpec((1, H), lambda: (0, 0)),
            pl.BlockSpec((H, 1), lambda: (0, 0)),
            pl.BlockSpec((1, 1), lambda: (0, 0)),
            pl.BlockSpec((N, 1), lambda: (0, 0)),
            pl.BlockSpec((H, L), lambda: (0, 0)),
            pl.BlockSpec((1, L), lambda: (0, 0)),
        ],
        out_specs=pl.BlockSpec((B, L), lambda: (0, 0)),
        out_shape=jax.ShapeDtypeStruct((B, L), jnp.float32),
    )(acc, y2, degp3, b2, Wg, bg, batch2d, Wm, bm)


def kernel(x, edge_index, batch, W1, b1, W2, b2, Wg, bg, Wm, bm, Wv, bv):
    del Wv, bv
    dst3 = edge_index[1].reshape(NTILES, NCH, CH)
    eidx4 = jnp.stack([edge_index[0].reshape(NTILES, ENCH, ECH),
                       edge_index[1].reshape(NTILES, ENCH, ECH)],
                      axis=2)

    ones_c = jnp.ones((CH,), jnp.float32)
    zeros_r = jnp.zeros((RPT,), jnp.float32)
    zeros_rows = jnp.zeros((RPT, H), jnp.float32)

    degp = _sc_degree(dst3, ones_c, zeros_r)
    degp3 = degp.reshape(2, NP, 1)

    y1 = _tc_scale_matmul(x, W1, degp3)
    acc1 = _sc_edge_accum(eidx4, y1, zeros_rows)
    y2 = _tc_conv1_out(acc1, y1, degp3,
                       b1.reshape(1, H), W2)
    acc2 = _sc_edge_accum(eidx4, y2, zeros_rows)
    mu = _tc_conv2_pool(acc2, y2, degp3, b2.reshape(1, H),
                        Wg, bg.reshape(1, 1),
                        batch.reshape(N, 1), Wm, bm.reshape(1, L))
    return mu


# speedup vs baseline: 1.0202x; 1.0194x over previous
import functools

import jax
import jax.numpy as jnp
from jax import lax
from jax.experimental import pallas as pl
from jax.experimental.pallas import tpu as pltpu
from jax.experimental.pallas import tpu_sc as plsc

N = 10000
E = 320000
D = 128
H = 128
L = 64
B = 16

NP = 10240
NTILES = 32
EPW = E // NTILES
CH = 80
NCH = EPW // CH
ECH = 80
ENCH = EPW // ECH
RPT = NP // 16

_MESH = plsc.VectorSubcoreMesh(core_axis_name="c", subcore_axis_name="s")


def _sc_degree(dst3, ones_c, zeros_r):
    @functools.partial(
        pl.kernel,
        mesh=_MESH,
        out_type=jax.ShapeDtypeStruct((2, NP), jnp.float32),
        scratch_types=[
            pltpu.VMEM((NCH, CH), jnp.int32),
            pltpu.VMEM((CH,), jnp.float32),
            pltpu.VMEM_SHARED((NP,), jnp.float32),
            pltpu.SemaphoreType.DMA,
            pltpu.SemaphoreType.DMA,
            pltpu.SemaphoreType.DMA,
            pltpu.SemaphoreType.DMA,
        ],
    )
    def k(dst_hbm, ones_hbm, zeros_hbm, out_hbm, idx_v, ones_v, acc_sh,
          sem0, sem1, sem2, sem3):
        c = lax.axis_index("c")
        s = lax.axis_index("s")
        w = c * 16 + s
        pltpu.sync_copy(ones_hbm, ones_v)
        pltpu.sync_copy(dst_hbm.at[w], idx_v)
        pltpu.sync_copy(zeros_hbm, acc_sh.at[pl.ds(s * RPT, RPT)])
        plsc.subcore_barrier()

        def scat(i, sem):
            pltpu.async_copy(ones_v, acc_sh.at[idx_v.at[i]], sem, add=True)

        def wait(i, sem):
            pltpu.make_async_copy(ones_v, acc_sh.at[idx_v.at[i]], sem).wait()

        sems = (sem0, sem1, sem2, sem3)
        for j in range(4):
            scat(j, sems[j])

        def body(g, carry):
            i = 4 * g
            for j in range(4):
                wait(i + j - 4, sems[j])
                scat(i + j, sems[j])
            return carry

        lax.fori_loop(1, (NCH - 1) // 4, body, 0)
        wait(NCH - 5, sem0)
        scat(NCH - 1, sem0)
        for j, i in ((1, NCH - 4), (2, NCH - 3), (3, NCH - 2), (0, NCH - 1)):
            wait(i, sems[j])
        plsc.subcore_barrier()
        pltpu.sync_copy(acc_sh.at[pl.ds(s * RPT, RPT)],
                        out_hbm.at[c, pl.ds(s * RPT, RPT)])

    return k(dst3, ones_c, zeros_r)


def _sc_edge_accum(idx_il, y, zeros_rows):
    NB = 4
    K = NB // 2
    NI = NB + K

    @functools.partial(
        pl.kernel,
        mesh=_MESH,
        out_type=jax.ShapeDtypeStruct((2, NP, H), jnp.float32),
        scratch_types=(
            [pltpu.VMEM((2, ECH), jnp.int32) for _ in range(NI)]
            + [pltpu.VMEM((ECH, H), jnp.float32) for _ in range(NB)]
            + [pltpu.VMEM_SHARED((NP, H), jnp.float32)]
            + [pltpu.SemaphoreType.DMA] * (2 * NB + NI)
        ),
    )
    def k(idx_hbm, y_hbm, zeros_hbm, out_hbm, *refs):
        idx = refs[0:NI]
        rows = refs[NI:NI + NB]
        acc_sh = refs[NI + NB]
        gsem = refs[NI + NB + 1:NI + 2 * NB + 1]
        scsem = refs[NI + 2 * NB + 1:NI + 3 * NB + 1]
        isem = refs[NI + 3 * NB + 1:2 * NI + 3 * NB + 1]
        c = lax.axis_index("c")
        s = lax.axis_index("s")
        w = c * 16 + s

        def load_idx(cd, m):
            pltpu.async_copy(idx_hbm.at[w, cd], idx[m], isem[m])

        def wait_idx(cd, m):
            pltpu.make_async_copy(idx_hbm.at[w, cd], idx[m], isem[m]).wait()

        def gather(j, m):
            pltpu.async_copy(y_hbm.at[idx[m].at[0]], rows[j], gsem[j])

        def wait_gather(j, m):
            pltpu.make_async_copy(
                y_hbm.at[idx[m].at[0]], rows[j], gsem[j]).wait()

        def scatter(j, m):
            pltpu.async_copy(rows[j], acc_sh.at[idx[m].at[1]],
                             scsem[j], add=True)

        def wait_scatter(j, m):
            pltpu.make_async_copy(
                rows[j], acc_sh.at[idx[m].at[1]], scsem[j]).wait()

        def step(cd, rr, do_scwait, do_load=True, do_gather=True):
            j = rr % NB
            wait_gather(j, rr % NI)
            scatter(j, rr % NI)
            jj = (j + K) % NB
            if do_scwait:
                wait_scatter(jj, (rr - K) % NI)
            if do_load:
                load_idx(cd + NB, (rr + NB) % NI)
            if do_gather:
                wait_idx(cd + K, (rr + K) % NI)
                gather(jj, (rr + K) % NI)

        for cc in range(NB):
            load_idx(cc, cc % NI)
        for j in range(K):
            wait_idx(j, j % NI)
            gather(j, j % NI)
        pltpu.sync_copy(zeros_hbm, acc_sh.at[pl.ds(s * RPT, RPT)])
        plsc.subcore_barrier()
        for cc in range(NB):
            step(cc, cc, do_scwait=cc >= K)

        U = 12
        T = (ENCH - K - NB) // U

        def body(h, carry):
            base = U * h + NB
            for r_off in range(U):
                step(base + r_off, NB + r_off, do_scwait=True)
            return carry

        lax.fori_loop(0, T, body, 0)

        for cc in range(NB + U * T, ENCH):
            if cc + K < ENCH:
                step(cc, cc, do_scwait=True, do_load=cc + NB < ENCH)
            else:
                wait_gather(cc % NB, cc % NI)
                scatter(cc % NB, cc % NI)
        for t in range(NB):
            cc = ENCH - NB + t
            wait_scatter(cc % NB, cc % NI)

        plsc.subcore_barrier()
        pltpu.sync_copy(acc_sh.at[pl.ds(s * RPT, RPT)],
                        out_hbm.at[c, pl.ds(s * RPT, RPT)])

    return k(idx_il, y, zeros_rows)


_RB = 1000
_NG = N // _RB


def _tc_scale_matmul(x, W, dinvb):
    def body(x_b, w_b, dinv_b, y_b):
        y_b[...] = jnp.dot(x_b[...], w_b[...],
                           preferred_element_type=jnp.float32) * dinv_b[...]

    return pl.pallas_call(
        body,
        grid=(_NG,),
        in_specs=[
            pl.BlockSpec((_RB, D), lambda i: (i, 0)),
            pl.BlockSpec((D, H), lambda i: (0, 0)),
            pl.BlockSpec((_RB, H), lambda i: (i, 0)),
        ],
        out_specs=pl.BlockSpec((_RB, H), lambda i: (i, 0)),
        out_shape=jax.ShapeDtypeStruct((N, H), jnp.float32),
    )(x, W, dinvb)


def _tc_conv1_out(acc, y1, dinvb, b1, W2):
    def body(acc_b, y1_b, dinv_b, b1_b, w2_b, y2_b):
        h = jnp.maximum(
            dinv_b[...] * (acc_b[0] + acc_b[1] + y1_b[...]) + b1_b[...],
            0.0)
        y2_b[...] = jnp.dot(h, w2_b[...],
                            preferred_element_type=jnp.float32) * dinv_b[...]

    return pl.pallas_call(
        body,
        grid=(_NG,),
        in_specs=[
            pl.BlockSpec((2, _RB, H), lambda i: (0, i, 0)),
            pl.BlockSpec((_RB, H), lambda i: (i, 0)),
            pl.BlockSpec((_RB, H), lambda i: (i, 0)),
            pl.BlockSpec((1, H), lambda i: (0, 0)),
            pl.BlockSpec((H, H), lambda i: (0, 0)),
        ],
        out_specs=pl.BlockSpec((_RB, H), lambda i: (i, 0)),
        out_shape=jax.ShapeDtypeStruct((N, H), jnp.float32),
    )(acc, y1, dinvb, b1, W2)


def _tc_conv2_pool(acc, y2, dinvb, b2, Wg, bg, batch2d, Wm, bm):
    def body(acc_r, y2_r, dinv_r, b2_r, wg_r, bg_r, batch_r, wm_r, bm_r,
             mu_r):
        h2 = (dinv_r[...] * (acc_r[0, 0:N, :] + acc_r[1, 0:N, :] + y2_r[...])
              + b2_r[...])
        g = jnp.dot(h2, wg_r[...],
                    preferred_element_type=jnp.float32) + bg_r[...]
        mask = (batch_r[...] ==
                lax.broadcasted_iota(jnp.int32, (N, B), 1))
        maskf = mask.astype(jnp.float32)
        gm = jnp.max(jnp.where(mask, g, -1e30), axis=0, keepdims=True)
        gmax_node = jnp.sum(maskf * gm, axis=1, keepdims=True)
        ge = jnp.exp(g - gmax_node)
        gs = jnp.sum(maskf * ge, axis=0, keepdims=True)
        gs_node = jnp.sum(maskf * gs, axis=1, keepdims=True)
        gate = ge / gs_node
        wgt = maskf * gate
        pooled = lax.dot_general(wgt, h2,
                                 (((0,), (0,)), ((), ())),
                                 preferred_element_type=jnp.float32)
        mu_r[...] = jnp.dot(pooled, wm_r[...],
                            preferred_element_type=jnp.float32) + bm_r[...]

    return pl.pallas_call(
        body,
        in_specs=[
            pl.BlockSpec((2, NP, H), lambda: (0, 0, 0)),
            pl.BlockSpec((N, H), lambda: (0, 0)),
            pl.BlockSpec((N, H), lambda: (0, 0)),
            pl.BlockSpec((1, H), lambda: (0, 0)),
            pl.BlockSpec((H, 1), lambda: (0, 0)),
            pl.BlockSpec((1, 1), lambda: (0, 0)),
            pl.BlockSpec((N, 1), lambda: (0, 0)),
            pl.BlockSpec((H, L), lambda: (0, 0)),
            pl.BlockSpec((1, L), lambda: (0, 0)),
        ],
        out_specs=pl.BlockSpec((B, L), lambda: (0, 0)),
        out_shape=jax.ShapeDtypeStruct((B, L), jnp.float32),
    )(acc, y2, dinvb, b2, Wg, bg, batch2d, Wm, bm)


def kernel(x, edge_index, batch, W1, b1, W2, b2, Wg, bg, Wm, bm, Wv, bv):
    del Wv, bv
    dst3 = edge_index[1].reshape(NTILES, NCH, CH)
    idx_il = jnp.stack(
        [edge_index[0].reshape(NTILES, ENCH, ECH),
         edge_index[1].reshape(NTILES, ENCH, ECH)], axis=2)

    ones_c = jnp.ones((CH,), jnp.float32)
    zeros_r = jnp.zeros((RPT,), jnp.float32)
    zeros_rows = jnp.zeros((RPT, H), jnp.float32)

    degp = _sc_degree(dst3, ones_c, zeros_r)
    dinvb = jnp.broadcast_to(
        lax.rsqrt(degp[0, :N] + degp[1, :N] + 1.0)[:, None], (N, H))

    y1 = _tc_scale_matmul(x, W1, dinvb)
    acc1 = _sc_edge_accum(idx_il, y1, zeros_rows)
    y2 = _tc_conv1_out(acc1, y1, dinvb,
                       b1.reshape(1, H), W2)
    acc2 = _sc_edge_accum(idx_il, y2, zeros_rows)
    mu = _tc_conv2_pool(acc2, y2, dinvb, b2.reshape(1, H),
                        Wg, bg.reshape(1, 1),
                        batch.reshape(N, 1), Wm, bm.reshape(1, L))
    return mu
